# baseline (speedup 1.0000x reference)
import functools

import jax
import jax.numpy as jnp
from jax import lax
from jax.experimental import pallas as pl
from jax.experimental.pallas import tpu as pltpu
from jax.experimental.pallas import tpu_sc as plsc

N_USER = 10000
N_ITEM = 10000
LATDIM = 128
HYPERNUM = 128
GNN_LAYER = 2
N_EDGES = 320000
BATCH = 4096
LEAKY = 0.5
TEMP = 1.0

NC = 2
NS = 16
L = 16

CHUNK = 96
KB = 12
NQ = 18
NT = KB * NQ
NCHUNKS = NS * NT
E_PAD = NCHUNKS * CHUNK
N_PAD = 10240
ROWS_PER_SUB = N_PAD // NS


def _lr(x):
    return jnp.maximum(LEAKY * x, x)


def _normalize(x):
    n = jnp.linalg.norm(x, axis=1, keepdims=True)
    return x / jnp.maximum(n, 1e-12)


def _spmm_body(tables, edp, vals, out,
               p0, p1, v0, v1, r0, r1, r2,
               ps0, ps1, vs0, vs1,
               g0, g1, g2, s0, s1, s2, acc_sh):
    c = lax.axis_index("c")
    s = lax.axis_index("s")
    packs = (p0, p1)
    vbufs = (v0, v1)
    psem = (ps0, ps1)
    vsem = (vs0, vs1)
    rows = (r0, r1, r2)
    gsem = (g0, g1, g2)
    ssem = (s0, s1, s2)
    t0 = s * NT

    def zero_rows(i, _):
        for k in range(8):
            r0[i, pl.ds(k * L, L)] = jnp.zeros((L,), jnp.float32)
        return 0
    lax.fori_loop(0, CHUNK, zero_rows, 0)
    base = s * ROWS_PER_SUB
    for q in range(ROWS_PER_SUB // CHUNK):
        pltpu.sync_copy(r0, acc_sh.at[pl.ds(base + q * CHUNK, CHUNK)])
    pltpu.sync_copy(r0.at[pl.ds(0, ROWS_PER_SUB % CHUNK)],
                    acc_sh.at[pl.ds(base + (ROWS_PER_SUB // CHUNK) * CHUNK,
                                    ROWS_PER_SUB % CHUNK)])
    plsc.subcore_barrier()

    def blk_start(q, p):
        pltpu.async_copy(edp.at[c, pl.ds(t0 + q * KB, KB)], packs[p], psem[p])
        pltpu.async_copy(vals.at[pl.ds(t0 + q * KB, KB)], vbufs[p], vsem[p])

    def blk_wait(q, p):
        pltpu.make_async_copy(edp.at[c, pl.ds(t0 + q * KB, KB)],
                              packs[p], psem[p]).wait()
        pltpu.make_async_copy(vals.at[pl.ds(t0 + q * KB, KB)],
                              vbufs[p], vsem[p]).wait()

    def g_start(kb, p, b):
        pltpu.async_copy(tables.at[packs[p].at[kb, 0]], rows[b], gsem[b])

    def g_wait(kb, p, b):
        pltpu.make_async_copy(tables.at[packs[p].at[kb, 0]],
                              rows[b], gsem[b]).wait()

    def a_start(kb, p, b):
        pltpu.async_copy(rows[b], acc_sh.at[packs[p].at[kb, 1]], ssem[b],
                         add=True)

    def a_wait(kb, p, b):
        pltpu.make_async_copy(rows[b], acc_sh.at[packs[p].at[kb, 1]],
                              ssem[b]).wait()

    blk_start(0, 0)
    blk_wait(0, 0)
    g_start(0, 0, 0)

    def chunk_step(q, qq, kb, last_pair):
        p = qq
        pn = 1 - qq
        b = kb % 3
        bn = (b + 1) % 3
        if kb >= 2:
            a_wait(kb - 2, p, bn)
        elif qq == 1:
            a_wait(kb + KB - 2, pn, bn)
        else:
            @pl.when(q > 0)
            def _():
                a_wait(kb + KB - 2, pn, bn)
        if kb == 2:
            if qq == 0:
                blk_start(q + 1, pn)
            else:
                @pl.when(last_pair == 0)
                def _():
                    blk_start(q + 1, pn)
        if kb < KB - 1:
            g_start(kb + 1, p, bn)
        elif qq == 0:
            blk_wait(q + 1, pn)
            g_start(0, pn, bn)
        else:
            @pl.when(last_pair == 0)
            def _():
                blk_wait(q + 1, pn)
                g_start(0, pn, bn)

        g_wait(kb, p, b)

        def scale_group(g, _):
            v16 = vbufs[p][kb, 0, pl.ds(g * L, L)]
            rb = rows[b]
            for j in range(L):
                vj = jnp.take(v16, jnp.full((L,), j, jnp.int32))
                e = g * L + j
                for k in range(8):
                    rb[e, pl.ds(k * L, L)] = rb[e, pl.ds(k * L, L)] * vj
            return 0
        lax.fori_loop(0, CHUNK // L, scale_group, 0)
        a_start(kb, p, b)

    def outer(m, _):
        last_pair = jnp.where(m == NQ // 2 - 1, 1, 0)
        for qq in range(2):
            q = m * 2 + qq
            for kb in range(KB):
                chunk_step(q, qq, kb, last_pair)
        return 0
    lax.fori_loop(0, NQ // 2, outer, 0)
    a_wait(KB - 2, 1, (KB - 2) % 3)
    a_wait(KB - 1, 1, (KB - 1) % 3)
    plsc.subcore_barrier()

    pltpu.sync_copy(acc_sh.at[pl.ds(base, ROWS_PER_SUB)],
                    out.at[c, pl.ds(base, ROWS_PER_SUB)])


@jax.jit
def _sc_spmm(flat_tables, edp, vals):
    mesh = plsc.VectorSubcoreMesh(core_axis_name="c", subcore_axis_name="s")
    return pl.kernel(
        _spmm_body,
        out_type=jax.ShapeDtypeStruct((2, N_PAD, LATDIM), jnp.float32),
        mesh=mesh,
        scratch_types=[
            pltpu.VMEM((KB, 2, CHUNK), jnp.int32),
            pltpu.VMEM((KB, 2, CHUNK), jnp.int32),
            pltpu.VMEM((KB, 1, CHUNK), jnp.float32),
            pltpu.VMEM((KB, 1, CHUNK), jnp.float32),
            pltpu.VMEM((CHUNK, LATDIM), jnp.float32),
            pltpu.VMEM((CHUNK, LATDIM), jnp.float32),
            pltpu.VMEM((CHUNK, LATDIM), jnp.float32),
            pltpu.SemaphoreType.DMA,
            pltpu.SemaphoreType.DMA,
            pltpu.SemaphoreType.DMA,
            pltpu.SemaphoreType.DMA,
            pltpu.SemaphoreType.DMA,
            pltpu.SemaphoreType.DMA,
            pltpu.SemaphoreType.DMA,
            pltpu.SemaphoreType.DMA,
            pltpu.SemaphoreType.DMA,
            pltpu.SemaphoreType.DMA,
            pltpu.VMEM_SHARED((N_PAD, LATDIM), jnp.float32),
        ],
    )(flat_tables, edp, vals)


def _mm_kernel(x_ref, w_ref, o_ref):
    o_ref[...] = jnp.dot(x_ref[...], w_ref[...],
                         preferred_element_type=jnp.float32)


def _pallas_mm(x, w):
    m, k = x.shape
    _, n = w.shape
    bm = 1000
    return pl.pallas_call(
        _mm_kernel,
        grid=(m // bm,),
        in_specs=[pl.BlockSpec((bm, k), lambda i: (i, 0)),
                  pl.BlockSpec((k, n), lambda i: (0, 0))],
        out_specs=pl.BlockSpec((bm, n), lambda i: (i, 0)),
        out_shape=jax.ShapeDtypeStruct((m, n), jnp.float32),
    )(x, w)


def _hyper_prop(lats, adj, W1, W2, W3):
    lat1 = _lr(adj.T @ lats)
    lat2 = _lr(lat1.T @ W1).T + lat1
    lat3 = _lr(lat2.T @ W2).T + lat2
    lat4 = _lr(lat3.T @ W3).T + lat3
    return _lr(adj @ lat4)


def _calc_ssl(h, g, m):
    pos = jnp.exp(jnp.sum(h * g, axis=1) / TEMP)
    neg = jnp.sum(jnp.exp(g @ h.T / TEMP) * m[None, :], axis=1)
    return jnp.sum(m * (-jnp.log(pos / (neg + 1e-08) + 1e-08)))


def kernel(uids, iids, edge_index, edge_vals, uEmbed0, iEmbed0, uhyper, ihyper, WU, WI, WT):
    row = edge_index[0]
    col = edge_index[1]
    uniq_u = jnp.unique(uids, size=BATCH, fill_value=0)
    uniq_i = jnp.unique(iids, size=BATCH, fill_value=0)
    present_u = jnp.zeros((N_USER,), dtype=jnp.bool_).at[uids].set(True)
    present_i = jnp.zeros((N_ITEM,), dtype=jnp.bool_).at[iids].set(True)
    num_u = jnp.sum(present_u)
    num_i = jnp.sum(present_i)
    mask_u = (jnp.arange(BATCH) < num_u).astype(jnp.float32)
    mask_i = (jnp.arange(BATCH) < num_i).astype(jnp.float32)

    padz = jnp.zeros((E_PAD - N_EDGES,), dtype=jnp.int32)
    colp = jnp.concatenate([col, padz]).reshape(NCHUNKS, CHUNK)
    rowp = jnp.concatenate([row, padz]).reshape(NCHUNKS, CHUNK)
    valp = jnp.concatenate([edge_vals, padz.astype(jnp.float32)]
                           ).reshape(NCHUNKS, 1, CHUNK)
    edp = jnp.stack([
        jnp.stack([colp, rowp], axis=1),
        jnp.stack([rowp + N_USER, colp], axis=1),
    ])

    uuHyper = _pallas_mm(uEmbed0, uhyper)
    iiHyper = _pallas_mm(iEmbed0, ihyper)
    ulats = [uEmbed0]
    ilats = [iEmbed0]
    gnnU, gnnI, hypU, hypI = [], [], [], []
    for i in range(GNN_LAYER):
        flat_tables = jnp.concatenate([ilats[-1], ulats[-1]], axis=0)
        acc = _sc_spmm(flat_tables, edp, valp)
        ulat = _lr(acc[0, :N_USER])
        ilat = _lr(acc[1, :N_ITEM])
        hU = _hyper_prop(ulats[-1], uuHyper, WU[i, 0], WU[i, 1], WU[i, 2])
        hI = _hyper_prop(ilats[-1], iiHyper, WI[i, 0], WI[i, 1], WI[i, 2])
        gnnU.append(ulat); gnnI.append(ilat); hypU.append(hU); hypI.append(hI)
        ulats.append(ulat + hU + ulats[-1])
        ilats.append(ilat + hI + ilats[-1])
    ulat = jnp.sum(jnp.stack(ulats), axis=0)
    ilat = jnp.sum(jnp.stack(ilats), axis=0)
    preds = jnp.sum(ulat[uids] * ilat[iids], axis=-1)
    ssl = 0.0
    for i in range(GNN_LAYER):
        pHU = _normalize(hypU[i][uniq_u]) @ WT[i]
        pGU = _normalize(gnnU[i][uniq_u])
        pHI = _normalize(hypI[i][uniq_i]) @ WT[i]
        pGI = _normalize(gnnI[i][uniq_i])
        ssl = ssl + _calc_ssl(pHU, pGU, mask_u) + _calc_ssl(pHI, pGI, mask_i)
    reg = (jnp.sum(jnp.square(uEmbed0)) + jnp.sum(jnp.square(iEmbed0))
           + jnp.sum(jnp.square(uhyper)) + jnp.sum(jnp.square(ihyper)))
    return (preds, ssl, reg)


# speedup vs baseline: 1.8123x; 1.8123x over previous
import functools

import jax
import jax.numpy as jnp
from jax import lax
from jax.experimental import pallas as pl
from jax.experimental.pallas import tpu as pltpu
from jax.experimental.pallas import tpu_sc as plsc

N_USER = 10000
N_ITEM = 10000
LATDIM = 128
HYPERNUM = 128
GNN_LAYER = 2
N_EDGES = 320000
BATCH = 4096
LEAKY = 0.5
TEMP = 1.0

NC = 2
NS = 16
L = 16

CHUNK = 112
NT = 180
NCHUNKS = NS * NT
E_PAD = NCHUNKS * CHUNK
N_PAD = 10240
ROWS_PER_SUB = N_PAD // NS


def _lr(x):
    return jnp.maximum(LEAKY * x, x)


def _normalize(x):
    n = jnp.linalg.norm(x, axis=1, keepdims=True)
    return x / jnp.maximum(n, 1e-12)


def _spmm_body(tables, edp, vals, out,
               p0, p1, p2, v0, v1, v2, r0, r1, r2,
               g0, g1, g2, s0, s1, s2, acc_sh):
    c = lax.axis_index("c")
    s = lax.axis_index("s")
    packs = (p0, p1, p2)
    vbufs = (v0, v1, v2)
    rows = (r0, r1, r2)
    gsem = (g0, g1, g2)
    ssem = (s0, s1, s2)
    t0 = s * NT

    def zero_rows(i, _):
        for k in range(8):
            r0[i, pl.ds(k * L, L)] = jnp.zeros((L,), jnp.float32)
        return 0
    lax.fori_loop(0, CHUNK, zero_rows, 0)
    base = s * ROWS_PER_SUB
    for q in range(ROWS_PER_SUB // CHUNK):
        pltpu.sync_copy(r0, acc_sh.at[pl.ds(base + q * CHUNK, CHUNK)])
    pltpu.sync_copy(r0.at[pl.ds(0, ROWS_PER_SUB % CHUNK)],
                    acc_sh.at[pl.ds(base + (ROWS_PER_SUB // CHUNK) * CHUNK,
                                    ROWS_PER_SUB % CHUNK)])
    plsc.subcore_barrier()

    def i_copy(t, b):
        pltpu.sync_copy(edp.at[c, t0 + t], packs[b])
        pltpu.sync_copy(vals.at[t0 + t], vbufs[b])

    def g_start(t, b):
        pltpu.async_copy(tables.at[packs[b].at[0]], rows[b], gsem[b])

    def g_wait(t, b):
        pltpu.make_async_copy(tables.at[packs[b].at[0]], rows[b], gsem[b]).wait()

    def a_start(t, b):
        pltpu.async_copy(rows[b], acc_sh.at[packs[b].at[1]], ssem[b], add=True)

    def a_wait(t, b):
        pltpu.make_async_copy(rows[b], acc_sh.at[packs[b].at[1]], ssem[b]).wait()

    i_copy(0, 0)
    g_start(0, 0)

    def outer(q, _):
        for b in range(3):
            t = q * 3 + b
            bn = (b + 1) % 3
            @pl.when(t >= 2)
            def _():
                a_wait(t - 2, bn)

            @pl.when(t + 1 < NT)
            def _():
                i_copy(t + 1, bn)
                g_start(t + 1, bn)

            g_wait(t, b)

            def scale_group(g, _):
                v16 = vbufs[b][0, pl.ds(g * L, L)]
                rb = rows[b]
                for j in range(L):
                    vj = jnp.take(v16, jnp.full((L,), j, jnp.int32))
                    e = g * L + j
                    for k in range(8):
                        rb[e, pl.ds(k * L, L)] = rb[e, pl.ds(k * L, L)] * vj
                return 0
            lax.fori_loop(0, CHUNK // L, scale_group, 0)
            a_start(t, b)
        return 0
    lax.fori_loop(0, NT // 3, outer, 0)
    a_wait(NT - 2, (NT - 2) % 3)
    a_wait(NT - 1, (NT - 1) % 3)
    plsc.subcore_barrier()

    pltpu.sync_copy(acc_sh.at[pl.ds(base, ROWS_PER_SUB)],
                    out.at[c, pl.ds(base, ROWS_PER_SUB)])


@jax.jit
def _sc_spmm(flat_tables, edp, vals):
    mesh = plsc.VectorSubcoreMesh(core_axis_name="c", subcore_axis_name="s")
    return pl.kernel(
        _spmm_body,
        out_type=jax.ShapeDtypeStruct((2, N_PAD, LATDIM), jnp.float32),
        mesh=mesh,
        scratch_types=[
            pltpu.VMEM((2, CHUNK), jnp.int32),
            pltpu.VMEM((2, CHUNK), jnp.int32),
            pltpu.VMEM((2, CHUNK), jnp.int32),
            pltpu.VMEM((1, CHUNK), jnp.float32),
            pltpu.VMEM((1, CHUNK), jnp.float32),
            pltpu.VMEM((1, CHUNK), jnp.float32),
            pltpu.VMEM((CHUNK, LATDIM), jnp.float32),
            pltpu.VMEM((CHUNK, LATDIM), jnp.float32),
            pltpu.VMEM((CHUNK, LATDIM), jnp.float32),
            pltpu.SemaphoreType.DMA,
            pltpu.SemaphoreType.DMA,
            pltpu.SemaphoreType.DMA,
            pltpu.SemaphoreType.DMA,
            pltpu.SemaphoreType.DMA,
            pltpu.SemaphoreType.DMA,
            pltpu.VMEM_SHARED((N_PAD, LATDIM), jnp.float32),
        ],
    )(flat_tables, edp, vals)


GB_PER_W = 10 * BATCH // (NC * NS)
GCHUNK = 128
GB_CH = GB_PER_W // GCHUNK


def _gather_body(table, idx, out, idx_v, rb0, rb1, gs0, gs1, os0, os1):
    c = lax.axis_index("c")
    s = lax.axis_index("s")
    w = s * NC + c
    base = w * GB_PER_W
    pltpu.sync_copy(idx.at[pl.ds(base, GB_PER_W)], idx_v)
    rbufs = (rb0, rb1)
    gsem = (gs0, gs1)
    osem = (os0, os1)

    def g_start(k, b):
        pltpu.async_copy(table.at[idx_v.at[pl.ds(k * GCHUNK, GCHUNK)]],
                         rbufs[b], gsem[b])

    def g_wait(k, b):
        pltpu.make_async_copy(table.at[idx_v.at[pl.ds(k * GCHUNK, GCHUNK)]],
                              rbufs[b], gsem[b]).wait()

    def o_start(k, b):
        pltpu.async_copy(rbufs[b], out.at[pl.ds(base + k * GCHUNK, GCHUNK)],
                         osem[b])

    def o_wait(k, b):
        pltpu.make_async_copy(rbufs[b], out.at[pl.ds(base + k * GCHUNK, GCHUNK)],
                              osem[b]).wait()

    g_start(0, 0)
    for k in range(GB_CH):
        b = k % 2
        g_wait(k, b)
        if k >= 1:
            o_wait(k - 1, 1 - b)
        if k + 1 < GB_CH:
            g_start(k + 1, 1 - b)
        o_start(k, b)
    o_wait(GB_CH - 1, (GB_CH - 1) % 2)


@jax.jit
def _sc_gather(table, idx):
    mesh = plsc.VectorSubcoreMesh(core_axis_name="c", subcore_axis_name="s")
    return pl.kernel(
        _gather_body,
        out_type=jax.ShapeDtypeStruct((10 * BATCH, LATDIM), jnp.float32),
        mesh=mesh,
        scratch_types=[
            pltpu.VMEM((GB_PER_W,), jnp.int32),
            pltpu.VMEM((GCHUNK, LATDIM), jnp.float32),
            pltpu.VMEM((GCHUNK, LATDIM), jnp.float32),
            pltpu.SemaphoreType.DMA,
            pltpu.SemaphoreType.DMA,
            pltpu.SemaphoreType.DMA,
            pltpu.SemaphoreType.DMA,
        ],
    )(table, idx)


RB = 2000
NBK = N_USER // RB


def _k0_body(e_ref, hw_ref, hyp_ref, reg_ref, acc):
    d = pl.program_id(0)
    i = pl.program_id(1)

    @pl.when((d == 0) & (i == 0))
    def _():
        acc[0] = 0.0
    eb = e_ref[0]
    hyp_ref[0] = jnp.dot(eb, hw_ref[0], preferred_element_type=jnp.float32)
    part = jnp.sum(eb * eb)

    @pl.when(i == 0)
    def _():
        acc[0] += jnp.sum(hw_ref[0] * hw_ref[0])
    acc[0] += part

    @pl.when((d == 1) & (i == NBK - 1))
    def _():
        reg_ref[...] = jnp.full((8, LATDIM), acc[0], jnp.float32)


@jax.jit
def _k0(E, Hw):
    return pl.pallas_call(
        _k0_body,
        grid=(2, NBK),
        in_specs=[
            pl.BlockSpec((1, RB, LATDIM), lambda d, i: (d, i, 0)),
            pl.BlockSpec((1, LATDIM, HYPERNUM), lambda d, i: (d, 0, 0)),
        ],
        out_specs=[
            pl.BlockSpec((1, RB, HYPERNUM), lambda d, i: (d, i, 0)),
            pl.BlockSpec((8, LATDIM), lambda d, i: (0, 0)),
        ],
        out_shape=[
            jax.ShapeDtypeStruct((2, N_USER, HYPERNUM), jnp.float32),
            jax.ShapeDtypeStruct((8, LATDIM), jnp.float32),
        ],
        scratch_shapes=[pltpu.SMEM((1,), jnp.float32)],
    )(E, Hw)


def _k1_body(lats_ref, hyp_ref, w_ref, lat4_ref, acc):
    i = pl.program_id(1)

    @pl.when(i == 0)
    def _():
        acc[...] = jnp.zeros((HYPERNUM, LATDIM), jnp.float32)
    dn = (((0,), (0,)), ((), ()))
    acc[...] += lax.dot_general(hyp_ref[0], lats_ref[0], dn,
                                preferred_element_type=jnp.float32)

    @pl.when(i == NBK - 1)
    def _():
        lat1 = _lr(acc[...])
        lat2 = _lr(lax.dot_general(lat1, w_ref[0, 0], dn,
                                   preferred_element_type=jnp.float32)).T + lat1
        lat3 = _lr(lax.dot_general(lat2, w_ref[0, 1], dn,
                                   preferred_element_type=jnp.float32)).T + lat2
        lat4 = _lr(lax.dot_general(lat3, w_ref[0, 2], dn,
                                   preferred_element_type=jnp.float32)).T + lat3
        lat4_ref[0] = lat4


@jax.jit
def _k1(lats, hyp, W):
    return pl.pallas_call(
        _k1_body,
        grid=(2, NBK),
        in_specs=[
            pl.BlockSpec((1, RB, LATDIM), lambda d, i: (d, i, 0)),
            pl.BlockSpec((1, RB, HYPERNUM), lambda d, i: (d, i, 0)),
            pl.BlockSpec((1, 3, HYPERNUM, HYPERNUM), lambda d, i: (d, 0, 0, 0)),
        ],
        out_specs=pl.BlockSpec((1, HYPERNUM, LATDIM), lambda d, i: (d, 0, 0)),
        out_shape=jax.ShapeDtypeStruct((2, HYPERNUM, LATDIM), jnp.float32),
        scratch_shapes=[pltpu.VMEM((HYPERNUM, LATDIM), jnp.float32)],
    )(lats, hyp, W)


def _k2_body(hyp_ref, lat4_ref, acc_ref, prev_ref, lats0_ref,
             hypo_ref, gnn_ref, latsn_ref, sums_ref):
    h = _lr(jnp.dot(hyp_ref[0], lat4_ref[0],
                    preferred_element_type=jnp.float32))
    g = _lr(acc_ref[0])
    p = prev_ref[0]
    ln = g + h + p
    hypo_ref[0] = h
    gnn_ref[0] = g
    latsn_ref[0] = ln
    if sums_ref is not None:
        sums_ref[0] = ln + p + lats0_ref[0]


@functools.partial(jax.jit, static_argnames=("with_sums",))
def _k2(hyp, lat4, acc, prev, lats0, with_sums=False):
    n_out = 4 if with_sums else 3
    body = _k2_body if with_sums else (
        lambda a, b, c, d, e, f, g, h: _k2_body(a, b, c, d, e, f, g, h, None))
    outs = pl.pallas_call(
        body,
        grid=(2, NBK),
        in_specs=[
            pl.BlockSpec((1, RB, HYPERNUM), lambda d, i: (d, i, 0)),
            pl.BlockSpec((1, HYPERNUM, LATDIM), lambda d, i: (d, 0, 0)),
            pl.BlockSpec((1, RB, LATDIM), lambda d, i: (d, i, 0)),
            pl.BlockSpec((1, RB, LATDIM), lambda d, i: (d, i, 0)),
            pl.BlockSpec((1, RB, LATDIM), lambda d, i: (d, i, 0)),
        ],
        out_specs=[pl.BlockSpec((1, RB, LATDIM), lambda d, i: (d, i, 0))
                   for _ in range(n_out)],
        out_shape=[jax.ShapeDtypeStruct((2, N_USER, LATDIM), jnp.float32)
                   for _ in range(n_out)],
    )(hyp, lat4, acc, prev, lats0)
    return outs


def _k3a_body(gh_ref, wt_ref, h_ref):
    x = gh_ref[0]
    nrm = jnp.sqrt(jnp.sum(x * x, axis=1, keepdims=True))
    xn = x / jnp.maximum(nrm, 1e-12)
    h_ref[0] = jnp.dot(xn, wt_ref[0], preferred_element_type=jnp.float32)


@jax.jit
def _k3a(ghyp, wts):
    return pl.pallas_call(
        _k3a_body,
        grid=(4,),
        in_specs=[
            pl.BlockSpec((1, BATCH, LATDIM), lambda j: (j, 0, 0)),
            pl.BlockSpec((1, LATDIM, LATDIM), lambda j: (j, 0, 0)),
        ],
        out_specs=pl.BlockSpec((1, BATCH, LATDIM), lambda j: (j, 0, 0)),
        out_shape=jax.ShapeDtypeStruct((4, BATCH, LATDIM), jnp.float32),
    )(ghyp, wts)


SSL_RB = 512
SSL_NB = BATCH // SSL_RB


def _k3b_body(h_ref, gg_ref, m_ref, ssl_ref, acc):
    j = pl.program_id(0)
    r = pl.program_id(1)

    @pl.when((j == 0) & (r == 0))
    def _():
        acc[0] = 0.0
    x = gg_ref[0]
    nrm = jnp.sqrt(jnp.sum(x * x, axis=1, keepdims=True))
    g = x / jnp.maximum(nrm, 1e-12)
    hfull = h_ref[0]
    dn = (((1,), (1,)), ((), ()))
    S = lax.dot_general(g, hfull, dn, preferred_element_type=jnp.float32)
    m = m_ref[0, 0]
    neg = jnp.sum(jnp.exp(S) * m[None, :], axis=1)
    hblk = h_ref[0, pl.ds(r * SSL_RB, SSL_RB), :]
    pos = jnp.exp(jnp.sum(hblk * g, axis=1))
    mrow = m_ref[0, 0, pl.ds(r * SSL_RB, SSL_RB)]
    acc[0] += jnp.sum(mrow * (-jnp.log(pos / (neg + 1e-08) + 1e-08)))

    @pl.when((j == 3) & (r == SSL_NB - 1))
    def _():
        ssl_ref[...] = jnp.full((8, LATDIM), acc[0], jnp.float32)


@jax.jit
def _k3b(h, ggnn, masks3):
    return pl.pallas_call(
        _k3b_body,
        grid=(4, SSL_NB),
        in_specs=[
            pl.BlockSpec((1, BATCH, LATDIM), lambda j, r: (j, 0, 0)),
            pl.BlockSpec((1, SSL_RB, LATDIM), lambda j, r: (j, r, 0)),
            pl.BlockSpec((1, 1, BATCH), lambda j, r: (j, 0, 0)),
        ],
        out_specs=pl.BlockSpec((8, LATDIM), lambda j, r: (0, 0)),
        out_shape=jax.ShapeDtypeStruct((8, LATDIM), jnp.float32),
        scratch_shapes=[pltpu.SMEM((1,), jnp.float32)],
    )(h, ggnn, masks3)


def _k4_body(u_ref, i_ref, o_ref):
    o_ref[...] = jnp.sum(u_ref[...] * i_ref[...], axis=-1)


@jax.jit
def _k4(u3, i3):
    return pl.pallas_call(
        _k4_body,
        grid=(1,),
        in_specs=[
            pl.BlockSpec((32, LATDIM, LATDIM), lambda i: (0, 0, 0)),
            pl.BlockSpec((32, LATDIM, LATDIM), lambda i: (0, 0, 0)),
        ],
        out_specs=pl.BlockSpec((32, LATDIM), lambda i: (0, 0)),
        out_shape=jax.ShapeDtypeStruct((32, LATDIM), jnp.float32),
    )(u3, i3)


def kernel(uids, iids, edge_index, edge_vals, uEmbed0, iEmbed0, uhyper, ihyper, WU, WI, WT):
    row = edge_index[0]
    col = edge_index[1]
    uniq_u = jnp.unique(uids, size=BATCH, fill_value=0)
    uniq_i = jnp.unique(iids, size=BATCH, fill_value=0)
    num_u = 1 + jnp.sum((uniq_u[1:] > uniq_u[:-1]).astype(jnp.int32))
    num_i = 1 + jnp.sum((uniq_i[1:] > uniq_i[:-1]).astype(jnp.int32))
    mask_u = (jnp.arange(BATCH) < num_u).astype(jnp.float32)
    mask_i = (jnp.arange(BATCH) < num_i).astype(jnp.float32)

    padz = jnp.zeros((E_PAD - N_EDGES,), dtype=jnp.int32)
    colp = jnp.concatenate([col, padz]).reshape(NCHUNKS, CHUNK)
    rowp = jnp.concatenate([row, padz]).reshape(NCHUNKS, CHUNK)
    valp = jnp.concatenate([edge_vals, padz.astype(jnp.float32)]
                           ).reshape(NCHUNKS, 1, CHUNK)
    edp = jnp.stack([
        jnp.stack([colp, rowp], axis=1),
        jnp.stack([rowp + N_USER, colp], axis=1),
    ])

    E0 = jnp.stack([uEmbed0, iEmbed0])
    Hw = jnp.stack([uhyper, ihyper])
    hyp, regp = _k0(E0, Hw)
    reg = regp[0, 0]

    lats = E0
    hypos, gnns = [], []
    sums = None
    for i in range(GNN_LAYER):
        flat_tables = jnp.concatenate([lats[1], lats[0]], axis=0)
        acc = _sc_spmm(flat_tables, edp, valp)
        W = jnp.stack([WU[i], WI[i]])
        lat4 = _k1(lats, hyp, W)
        outs = _k2(hyp, lat4, acc, lats, E0, with_sums=(i == GNN_LAYER - 1))
        if i == GNN_LAYER - 1:
            hypo, gnn, lats, sums = outs
        else:
            hypo, gnn, lats = outs
        hypos.append(hypo)
        gnns.append(gnn)

    table = jnp.concatenate(
        [hypos[0].reshape(-1, LATDIM), hypos[1].reshape(-1, LATDIM),
         gnns[0].reshape(-1, LATDIM), gnns[1].reshape(-1, LATDIM),
         sums.reshape(-1, LATDIM)], axis=0)
    idx_all = jnp.concatenate([
        uniq_u, uniq_i + N_USER,
        uniq_u + 2 * N_USER, uniq_i + 3 * N_USER,
        uniq_u + 4 * N_USER, uniq_i + 5 * N_USER,
        uniq_u + 6 * N_USER, uniq_i + 7 * N_USER,
        uids + 8 * N_USER, iids + 9 * N_USER,
    ]).astype(jnp.int32)
    rowsg = _sc_gather(table, idx_all).reshape(10, BATCH, LATDIM)

    ghyp = rowsg[0:4]
    ggnn = rowsg[4:8]
    wts = jnp.stack([WT[0], WT[0], WT[1], WT[1]])
    masks3 = jnp.stack([mask_u, mask_i, mask_u, mask_i])[:, None, :]
    h = _k3a(ghyp, wts)
    sslp = _k3b(h, ggnn, masks3)
    ssl = sslp[0, 0]

    preds = _k4(rowsg[8].reshape(32, LATDIM, LATDIM),
                rowsg[9].reshape(32, LATDIM, LATDIM)).reshape(BATCH)
    return (preds, ssl, reg)


# speedup vs baseline: 1.9801x; 1.0926x over previous
import functools

import jax
import jax.numpy as jnp
from jax import lax
from jax.experimental import pallas as pl
from jax.experimental.pallas import tpu as pltpu
from jax.experimental.pallas import tpu_sc as plsc

N_USER = 10000
N_ITEM = 10000
LATDIM = 128
HYPERNUM = 128
GNN_LAYER = 2
N_EDGES = 320000
BATCH = 4096
LEAKY = 0.5
TEMP = 1.0

NC = 2
NS = 16
L = 16

CHUNK = 112
NT = 180
NCHUNKS = NS * NT
E_PAD = NCHUNKS * CHUNK
N_PAD = 10240
ROWS_PER_SUB = N_PAD // NS


def _lr(x):
    return jnp.maximum(LEAKY * x, x)


def _normalize(x):
    n = jnp.linalg.norm(x, axis=1, keepdims=True)
    return x / jnp.maximum(n, 1e-12)


def _spmm_body(tables, edp, vals, out,
               p0, p1, p2, v0, v1, v2, r0, r1, r2,
               g0, g1, g2, s0, s1, s2, i0, i1, i2, acc_sh):
    c = lax.axis_index("c")
    s = lax.axis_index("s")
    packs = (p0, p1, p2)
    vbufs = (v0, v1, v2)
    rows = (r0, r1, r2)
    gsem = (g0, g1, g2)
    ssem = (s0, s1, s2)
    isem = (i0, i1, i2)
    t0 = s * NT

    def zero_rows(i, _):
        for k in range(8):
            r0[i, pl.ds(k * L, L)] = jnp.zeros((L,), jnp.float32)
        return 0
    lax.fori_loop(0, CHUNK, zero_rows, 0)
    base = s * ROWS_PER_SUB
    for q in range(ROWS_PER_SUB // CHUNK):
        pltpu.sync_copy(r0, acc_sh.at[pl.ds(base + q * CHUNK, CHUNK)])
    pltpu.sync_copy(r0.at[pl.ds(0, ROWS_PER_SUB % CHUNK)],
                    acc_sh.at[pl.ds(base + (ROWS_PER_SUB // CHUNK) * CHUNK,
                                    ROWS_PER_SUB % CHUNK)])
    plsc.subcore_barrier()

    def i_copy(t, b):
        pltpu.async_copy(edp.at[c, t0 + t], packs[b], isem[b])
        pltpu.async_copy(vals.at[t0 + t], vbufs[b], isem[b])
        pltpu.make_async_copy(edp.at[c, t0 + t], packs[b], isem[b]).wait()
        pltpu.make_async_copy(vals.at[t0 + t], vbufs[b], isem[b]).wait()

    def g_start(t, b):
        pltpu.async_copy(tables.at[packs[b].at[0]], rows[b], gsem[b])

    def g_wait(t, b):
        pltpu.make_async_copy(tables.at[packs[b].at[0]], rows[b], gsem[b]).wait()

    def a_start(t, b):
        pltpu.async_copy(rows[b], acc_sh.at[packs[b].at[1]], ssem[b], add=True)

    def a_wait(t, b):
        pltpu.make_async_copy(rows[b], acc_sh.at[packs[b].at[1]], ssem[b]).wait()

    i_copy(0, 0)
    g_start(0, 0)

    def outer(q, _):
        for b in range(3):
            t = q * 3 + b
            bn = (b + 1) % 3
            @pl.when(t >= 2)
            def _():
                a_wait(t - 2, bn)

            @pl.when(t + 1 < NT)
            def _():
                i_copy(t + 1, bn)
                g_start(t + 1, bn)

            g_wait(t, b)

            def scale_group(g, _):
                v16 = vbufs[b][0, pl.ds(g * L, L)]
                rb = rows[b]
                for j in range(L):
                    vj = jnp.take(v16, jnp.full((L,), j, jnp.int32))
                    e = g * L + j
                    for k in range(8):
                        rb[e, pl.ds(k * L, L)] = rb[e, pl.ds(k * L, L)] * vj
                return 0
            lax.fori_loop(0, CHUNK // L, scale_group, 0)
            a_start(t, b)
        return 0
    lax.fori_loop(0, NT // 3, outer, 0)
    a_wait(NT - 2, (NT - 2) % 3)
    a_wait(NT - 1, (NT - 1) % 3)
    plsc.subcore_barrier()

    pltpu.sync_copy(acc_sh.at[pl.ds(base, ROWS_PER_SUB)],
                    out.at[c, pl.ds(base, ROWS_PER_SUB)])


@jax.jit
def _sc_spmm(flat_tables, edp, vals):
    mesh = plsc.VectorSubcoreMesh(core_axis_name="c", subcore_axis_name="s")
    return pl.kernel(
        _spmm_body,
        out_type=jax.ShapeDtypeStruct((2, N_PAD, LATDIM), jnp.float32),
        mesh=mesh,
        scratch_types=[
            pltpu.VMEM((2, CHUNK), jnp.int32),
            pltpu.VMEM((2, CHUNK), jnp.int32),
            pltpu.VMEM((2, CHUNK), jnp.int32),
            pltpu.VMEM((1, CHUNK), jnp.float32),
            pltpu.VMEM((1, CHUNK), jnp.float32),
            pltpu.VMEM((1, CHUNK), jnp.float32),
            pltpu.VMEM((CHUNK, LATDIM), jnp.float32),
            pltpu.VMEM((CHUNK, LATDIM), jnp.float32),
            pltpu.VMEM((CHUNK, LATDIM), jnp.float32),
            pltpu.SemaphoreType.DMA,
            pltpu.SemaphoreType.DMA,
            pltpu.SemaphoreType.DMA,
            pltpu.SemaphoreType.DMA,
            pltpu.SemaphoreType.DMA,
            pltpu.SemaphoreType.DMA,
            pltpu.SemaphoreType.DMA,
            pltpu.SemaphoreType.DMA,
            pltpu.SemaphoreType.DMA,
            pltpu.VMEM_SHARED((N_PAD, LATDIM), jnp.float32),
        ],
    )(flat_tables, edp, vals)


GB_PER_W = 10 * BATCH // (NC * NS)
GCHUNK = 128
GB_CH = GB_PER_W // GCHUNK


def _gather_body(table, idx, out, idx_v, rb0, rb1, gs0, gs1, os0, os1):
    c = lax.axis_index("c")
    s = lax.axis_index("s")
    w = s * NC + c
    base = w * GB_PER_W
    pltpu.sync_copy(idx.at[pl.ds(base, GB_PER_W)], idx_v)
    rbufs = (rb0, rb1)
    gsem = (gs0, gs1)
    osem = (os0, os1)

    def g_start(k, b):
        pltpu.async_copy(table.at[idx_v.at[pl.ds(k * GCHUNK, GCHUNK)]],
                         rbufs[b], gsem[b])

    def g_wait(k, b):
        pltpu.make_async_copy(table.at[idx_v.at[pl.ds(k * GCHUNK, GCHUNK)]],
                              rbufs[b], gsem[b]).wait()

    def o_start(k, b):
        pltpu.async_copy(rbufs[b], out.at[pl.ds(base + k * GCHUNK, GCHUNK)],
                         osem[b])

    def o_wait(k, b):
        pltpu.make_async_copy(rbufs[b], out.at[pl.ds(base + k * GCHUNK, GCHUNK)],
                              osem[b]).wait()

    g_start(0, 0)
    for k in range(GB_CH):
        b = k % 2
        g_wait(k, b)
        if k >= 1:
            o_wait(k - 1, 1 - b)
        if k + 1 < GB_CH:
            g_start(k + 1, 1 - b)
        o_start(k, b)
    o_wait(GB_CH - 1, (GB_CH - 1) % 2)


@jax.jit
def _sc_gather(table, idx):
    mesh = plsc.VectorSubcoreMesh(core_axis_name="c", subcore_axis_name="s")
    return pl.kernel(
        _gather_body,
        out_type=jax.ShapeDtypeStruct((10 * BATCH, LATDIM), jnp.float32),
        mesh=mesh,
        scratch_types=[
            pltpu.VMEM((GB_PER_W,), jnp.int32),
            pltpu.VMEM((GCHUNK, LATDIM), jnp.float32),
            pltpu.VMEM((GCHUNK, LATDIM), jnp.float32),
            pltpu.SemaphoreType.DMA,
            pltpu.SemaphoreType.DMA,
            pltpu.SemaphoreType.DMA,
            pltpu.SemaphoreType.DMA,
        ],
    )(table, idx)


RB = 2000
NBK = N_USER // RB


def _k0_body(e_ref, hw_ref, hyp_ref, reg_ref, acc):
    d = pl.program_id(0)
    i = pl.program_id(1)

    @pl.when((d == 0) & (i == 0))
    def _():
        acc[0] = 0.0
    eb = e_ref[0]
    hyp_ref[0] = jnp.dot(eb, hw_ref[0], preferred_element_type=jnp.float32)
    part = jnp.sum(eb * eb)

    @pl.when(i == 0)
    def _():
        acc[0] += jnp.sum(hw_ref[0] * hw_ref[0])
    acc[0] += part

    @pl.when((d == 1) & (i == NBK - 1))
    def _():
        reg_ref[...] = jnp.full((8, LATDIM), acc[0], jnp.float32)


@jax.jit
def _k0(E, Hw):
    return pl.pallas_call(
        _k0_body,
        grid=(2, NBK),
        in_specs=[
            pl.BlockSpec((1, RB, LATDIM), lambda d, i: (d, i, 0)),
            pl.BlockSpec((1, LATDIM, HYPERNUM), lambda d, i: (d, 0, 0)),
        ],
        out_specs=[
            pl.BlockSpec((1, RB, HYPERNUM), lambda d, i: (d, i, 0)),
            pl.BlockSpec((8, LATDIM), lambda d, i: (0, 0)),
        ],
        out_shape=[
            jax.ShapeDtypeStruct((2, N_USER, HYPERNUM), jnp.float32),
            jax.ShapeDtypeStruct((8, LATDIM), jnp.float32),
        ],
        scratch_shapes=[pltpu.SMEM((1,), jnp.float32)],
    )(E, Hw)


def _k1_body(lats_ref, hyp_ref, w_ref, lat4_ref, acc):
    i = pl.program_id(1)

    @pl.when(i == 0)
    def _():
        acc[...] = jnp.zeros((HYPERNUM, LATDIM), jnp.float32)
    dn = (((0,), (0,)), ((), ()))
    acc[...] += lax.dot_general(hyp_ref[0], lats_ref[0], dn,
                                preferred_element_type=jnp.float32)

    @pl.when(i == NBK - 1)
    def _():
        lat1 = _lr(acc[...])
        lat2 = _lr(lax.dot_general(lat1, w_ref[0, 0], dn,
                                   preferred_element_type=jnp.float32)).T + lat1
        lat3 = _lr(lax.dot_general(lat2, w_ref[0, 1], dn,
                                   preferred_element_type=jnp.float32)).T + lat2
        lat4 = _lr(lax.dot_general(lat3, w_ref[0, 2], dn,
                                   preferred_element_type=jnp.float32)).T + lat3
        lat4_ref[0] = lat4


@jax.jit
def _k1(lats, hyp, W):
    return pl.pallas_call(
        _k1_body,
        grid=(2, NBK),
        in_specs=[
            pl.BlockSpec((1, RB, LATDIM), lambda d, i: (d, i, 0)),
            pl.BlockSpec((1, RB, HYPERNUM), lambda d, i: (d, i, 0)),
            pl.BlockSpec((1, 3, HYPERNUM, HYPERNUM), lambda d, i: (d, 0, 0, 0)),
        ],
        out_specs=pl.BlockSpec((1, HYPERNUM, LATDIM), lambda d, i: (d, 0, 0)),
        out_shape=jax.ShapeDtypeStruct((2, HYPERNUM, LATDIM), jnp.float32),
        scratch_shapes=[pltpu.VMEM((HYPERNUM, LATDIM), jnp.float32)],
    )(lats, hyp, W)


def _k2_body(hyp_ref, lat4_ref, acc_ref, prev_ref, lats0_ref,
             hypo_ref, gnn_ref, latsn_ref, sums_ref):
    h = _lr(jnp.dot(hyp_ref[0], lat4_ref[0],
                    preferred_element_type=jnp.float32))
    g = _lr(acc_ref[0])
    p = prev_ref[0]
    ln = g + h + p
    hypo_ref[0] = h
    gnn_ref[0] = g
    latsn_ref[0] = ln
    if sums_ref is not None:
        sums_ref[0] = ln + p + lats0_ref[0]


@functools.partial(jax.jit, static_argnames=("with_sums",))
def _k2(hyp, lat4, acc, prev, lats0, with_sums=False):
    n_out = 4 if with_sums else 3
    body = _k2_body if with_sums else (
        lambda a, b, c, d, e, f, g, h: _k2_body(a, b, c, d, e, f, g, h, None))
    outs = pl.pallas_call(
        body,
        grid=(2, NBK),
        in_specs=[
            pl.BlockSpec((1, RB, HYPERNUM), lambda d, i: (d, i, 0)),
            pl.BlockSpec((1, HYPERNUM, LATDIM), lambda d, i: (d, 0, 0)),
            pl.BlockSpec((1, RB, LATDIM), lambda d, i: (d, i, 0)),
            pl.BlockSpec((1, RB, LATDIM), lambda d, i: (d, i, 0)),
            pl.BlockSpec((1, RB, LATDIM), lambda d, i: (d, i, 0)),
        ],
        out_specs=[pl.BlockSpec((1, RB, LATDIM), lambda d, i: (d, i, 0))
                   for _ in range(n_out)],
        out_shape=[jax.ShapeDtypeStruct((2, N_USER, LATDIM), jnp.float32)
                   for _ in range(n_out)],
    )(hyp, lat4, acc, prev, lats0)
    return outs


def _k3a_body(gh_ref, wt_ref, h_ref):
    x = gh_ref[0]
    nrm = jnp.sqrt(jnp.sum(x * x, axis=1, keepdims=True))
    xn = x / jnp.maximum(nrm, 1e-12)
    h_ref[0] = jnp.dot(xn, wt_ref[0], preferred_element_type=jnp.float32)


@jax.jit
def _k3a(ghyp, wts):
    return pl.pallas_call(
        _k3a_body,
        grid=(4,),
        in_specs=[
            pl.BlockSpec((1, BATCH, LATDIM), lambda j: (j, 0, 0)),
            pl.BlockSpec((1, LATDIM, LATDIM), lambda j: (j, 0, 0)),
        ],
        out_specs=pl.BlockSpec((1, BATCH, LATDIM), lambda j: (j, 0, 0)),
        out_shape=jax.ShapeDtypeStruct((4, BATCH, LATDIM), jnp.float32),
    )(ghyp, wts)


SSL_RB = 512
SSL_NB = BATCH // SSL_RB


def _k3b_body(h_ref, gg_ref, m_ref, ssl_ref, acc):
    j = pl.program_id(0)
    r = pl.program_id(1)

    @pl.when((j == 0) & (r == 0))
    def _():
        acc[0] = 0.0
    x = gg_ref[0]
    nrm = jnp.sqrt(jnp.sum(x * x, axis=1, keepdims=True))
    g = x / jnp.maximum(nrm, 1e-12)
    hfull = h_ref[0]
    dn = (((1,), (1,)), ((), ()))
    S = lax.dot_general(g, hfull, dn, preferred_element_type=jnp.float32)
    m = m_ref[0, 0]
    neg = jnp.sum(jnp.exp(S) * m[None, :], axis=1)
    hblk = h_ref[0, pl.ds(r * SSL_RB, SSL_RB), :]
    pos = jnp.exp(jnp.sum(hblk * g, axis=1))
    mrow = m_ref[0, 0, pl.ds(r * SSL_RB, SSL_RB)]
    acc[0] += jnp.sum(mrow * (-jnp.log(pos / (neg + 1e-08) + 1e-08)))

    @pl.when((j == 3) & (r == SSL_NB - 1))
    def _():
        ssl_ref[...] = jnp.full((8, LATDIM), acc[0], jnp.float32)


@jax.jit
def _k3b(h, ggnn, masks3):
    return pl.pallas_call(
        _k3b_body,
        grid=(4, SSL_NB),
        in_specs=[
            pl.BlockSpec((1, BATCH, LATDIM), lambda j, r: (j, 0, 0)),
            pl.BlockSpec((1, SSL_RB, LATDIM), lambda j, r: (j, r, 0)),
            pl.BlockSpec((1, 1, BATCH), lambda j, r: (j, 0, 0)),
        ],
        out_specs=pl.BlockSpec((8, LATDIM), lambda j, r: (0, 0)),
        out_shape=jax.ShapeDtypeStruct((8, LATDIM), jnp.float32),
        scratch_shapes=[pltpu.SMEM((1,), jnp.float32)],
    )(h, ggnn, masks3)


def _k4_body(u_ref, i_ref, o_ref):
    o_ref[...] = jnp.sum(u_ref[...] * i_ref[...], axis=-1)


@jax.jit
def _k4(u3, i3):
    return pl.pallas_call(
        _k4_body,
        grid=(1,),
        in_specs=[
            pl.BlockSpec((32, LATDIM, LATDIM), lambda i: (0, 0, 0)),
            pl.BlockSpec((32, LATDIM, LATDIM), lambda i: (0, 0, 0)),
        ],
        out_specs=pl.BlockSpec((32, LATDIM), lambda i: (0, 0)),
        out_shape=jax.ShapeDtypeStruct((32, LATDIM), jnp.float32),
    )(u3, i3)


def kernel(uids, iids, edge_index, edge_vals, uEmbed0, iEmbed0, uhyper, ihyper, WU, WI, WT):
    row = edge_index[0]
    col = edge_index[1]
    uniq_u = jnp.unique(uids, size=BATCH, fill_value=0)
    uniq_i = jnp.unique(iids, size=BATCH, fill_value=0)
    num_u = 1 + jnp.sum((uniq_u[1:] > uniq_u[:-1]).astype(jnp.int32))
    num_i = 1 + jnp.sum((uniq_i[1:] > uniq_i[:-1]).astype(jnp.int32))
    mask_u = (jnp.arange(BATCH) < num_u).astype(jnp.float32)
    mask_i = (jnp.arange(BATCH) < num_i).astype(jnp.float32)

    padz = jnp.zeros((E_PAD - N_EDGES,), dtype=jnp.int32)
    colp = jnp.concatenate([col, padz]).reshape(NCHUNKS, CHUNK)
    rowp = jnp.concatenate([row, padz]).reshape(NCHUNKS, CHUNK)
    valp = jnp.concatenate([edge_vals, padz.astype(jnp.float32)]
                           ).reshape(NCHUNKS, 1, CHUNK)
    edp = jnp.stack([
        jnp.stack([colp, rowp], axis=1),
        jnp.stack([rowp + N_USER, colp], axis=1),
    ])

    E0 = jnp.stack([uEmbed0, iEmbed0])
    Hw = jnp.stack([uhyper, ihyper])
    hyp, regp = _k0(E0, Hw)
    reg = regp[0, 0]

    lats = E0
    hypos, gnns = [], []
    sums = None
    for i in range(GNN_LAYER):
        flat_tables = jnp.concatenate([lats[1], lats[0]], axis=0)
        acc = _sc_spmm(flat_tables, edp, valp)
        W = jnp.stack([WU[i], WI[i]])
        lat4 = _k1(lats, hyp, W)
        outs = _k2(hyp, lat4, acc, lats, E0, with_sums=(i == GNN_LAYER - 1))
        if i == GNN_LAYER - 1:
            hypo, gnn, lats, sums = outs
        else:
            hypo, gnn, lats = outs
        hypos.append(hypo)
        gnns.append(gnn)

    table = jnp.concatenate(
        [hypos[0].reshape(-1, LATDIM), hypos[1].reshape(-1, LATDIM),
         gnns[0].reshape(-1, LATDIM), gnns[1].reshape(-1, LATDIM),
         sums.reshape(-1, LATDIM)], axis=0)
    idx_all = jnp.concatenate([
        uniq_u, uniq_i + N_USER,
        uniq_u + 2 * N_USER, uniq_i + 3 * N_USER,
        uniq_u + 4 * N_USER, uniq_i + 5 * N_USER,
        uniq_u + 6 * N_USER, uniq_i + 7 * N_USER,
        uids + 8 * N_USER, iids + 9 * N_USER,
    ]).astype(jnp.int32)
    rowsg = _sc_gather(table, idx_all).reshape(10, BATCH, LATDIM)

    ghyp = rowsg[0:4]
    ggnn = rowsg[4:8]
    wts = jnp.stack([WT[0], WT[0], WT[1], WT[1]])
    masks3 = jnp.stack([mask_u, mask_i, mask_u, mask_i])[:, None, :]
    h = _k3a(ghyp, wts)
    sslp = _k3b(h, ggnn, masks3)
    ssl = sslp[0, 0]

    preds = _k4(rowsg[8].reshape(32, LATDIM, LATDIM),
                rowsg[9].reshape(32, LATDIM, LATDIM)).reshape(BATCH)
    return (preds, ssl, reg)


# speedup vs baseline: 1.9809x; 1.0004x over previous
import functools

import jax
import jax.numpy as jnp
from jax import lax
from jax.experimental import pallas as pl
from jax.experimental.pallas import tpu as pltpu
from jax.experimental.pallas import tpu_sc as plsc

N_USER = 10000
N_ITEM = 10000
LATDIM = 128
HYPERNUM = 128
GNN_LAYER = 2
N_EDGES = 320000
BATCH = 4096
LEAKY = 0.5
TEMP = 1.0

NC = 2
NS = 16
L = 16

CHUNK = 112
NT = 180
NCHUNKS = NS * NT
E_PAD = NCHUNKS * CHUNK
N_PAD = 10240
ROWS_PER_SUB = N_PAD // NS


def _lr(x):
    return jnp.maximum(LEAKY * x, x)


def _normalize(x):
    n = jnp.linalg.norm(x, axis=1, keepdims=True)
    return x / jnp.maximum(n, 1e-12)


def _spmm_body(tables, edp, vals, out,
               p0, p1, p2, v0, v1, v2, r0, r1, r2,
               g0, g1, g2, s0, s1, s2, i0, i1, i2, acc_sh):
    c = lax.axis_index("c")
    s = lax.axis_index("s")
    packs = (p0, p1, p2)
    vbufs = (v0, v1, v2)
    rows = (r0, r1, r2)
    gsem = (g0, g1, g2)
    ssem = (s0, s1, s2)
    isem = (i0, i1, i2)
    t0 = s * NT

    def zero_rows(i, _):
        for k in range(8):
            r0[i, pl.ds(k * L, L)] = jnp.zeros((L,), jnp.float32)
        return 0
    lax.fori_loop(0, CHUNK, zero_rows, 0)
    base = s * ROWS_PER_SUB
    for q in range(ROWS_PER_SUB // CHUNK):
        pltpu.sync_copy(r0, acc_sh.at[pl.ds(base + q * CHUNK, CHUNK)])
    pltpu.sync_copy(r0.at[pl.ds(0, ROWS_PER_SUB % CHUNK)],
                    acc_sh.at[pl.ds(base + (ROWS_PER_SUB // CHUNK) * CHUNK,
                                    ROWS_PER_SUB % CHUNK)])
    plsc.subcore_barrier()

    def i_copy(t, b):
        pltpu.async_copy(edp.at[c, t0 + t], packs[b], isem[b])
        pltpu.async_copy(vals.at[t0 + t], vbufs[b], isem[b])
        pltpu.make_async_copy(edp.at[c, t0 + t], packs[b], isem[b]).wait()
        pltpu.make_async_copy(vals.at[t0 + t], vbufs[b], isem[b]).wait()

    def g_start(t, b):
        pltpu.async_copy(tables.at[packs[b].at[0]], rows[b], gsem[b])

    def g_wait(t, b):
        pltpu.make_async_copy(tables.at[packs[b].at[0]], rows[b], gsem[b]).wait()

    def a_start(t, b):
        pltpu.async_copy(rows[b], acc_sh.at[packs[b].at[1]], ssem[b], add=True)

    def a_wait(t, b):
        pltpu.make_async_copy(rows[b], acc_sh.at[packs[b].at[1]], ssem[b]).wait()

    i_copy(0, 0)
    g_start(0, 0)

    def outer(q, _):
        for b in range(3):
            t = q * 3 + b
            bn = (b + 1) % 3
            @pl.when(t >= 2)
            def _():
                a_wait(t - 2, bn)

            @pl.when(t + 1 < NT)
            def _():
                i_copy(t + 1, bn)
                g_start(t + 1, bn)

            g_wait(t, b)

            def scale_group(g, _):
                v16 = vbufs[b][0, pl.ds(g * L, L)]
                rb = rows[b]
                for j in range(L):
                    vj = jnp.take(v16, jnp.full((L,), j, jnp.int32))
                    e = g * L + j
                    for k in range(8):
                        rb[e, pl.ds(k * L, L)] = rb[e, pl.ds(k * L, L)] * vj
                return 0
            lax.fori_loop(0, CHUNK // L, scale_group, 0)
            a_start(t, b)
        return 0
    lax.fori_loop(0, NT // 3, outer, 0)
    a_wait(NT - 2, (NT - 2) % 3)
    a_wait(NT - 1, (NT - 1) % 3)
    plsc.subcore_barrier()

    pltpu.sync_copy(acc_sh.at[pl.ds(base, ROWS_PER_SUB)],
                    out.at[c, pl.ds(base, ROWS_PER_SUB)])


@jax.jit
def _sc_spmm(flat_tables, edp, vals):
    mesh = plsc.VectorSubcoreMesh(core_axis_name="c", subcore_axis_name="s")
    return pl.kernel(
        _spmm_body,
        out_type=jax.ShapeDtypeStruct((2, N_PAD, LATDIM), jnp.float32),
        mesh=mesh,
        scratch_types=[
            pltpu.VMEM((2, CHUNK), jnp.int32),
            pltpu.VMEM((2, CHUNK), jnp.int32),
            pltpu.VMEM((2, CHUNK), jnp.int32),
            pltpu.VMEM((1, CHUNK), jnp.float32),
            pltpu.VMEM((1, CHUNK), jnp.float32),
            pltpu.VMEM((1, CHUNK), jnp.float32),
            pltpu.VMEM((CHUNK, LATDIM), jnp.float32),
            pltpu.VMEM((CHUNK, LATDIM), jnp.float32),
            pltpu.VMEM((CHUNK, LATDIM), jnp.float32),
            pltpu.SemaphoreType.DMA,
            pltpu.SemaphoreType.DMA,
            pltpu.SemaphoreType.DMA,
            pltpu.SemaphoreType.DMA,
            pltpu.SemaphoreType.DMA,
            pltpu.SemaphoreType.DMA,
            pltpu.SemaphoreType.DMA,
            pltpu.SemaphoreType.DMA,
            pltpu.SemaphoreType.DMA,
            pltpu.VMEM_SHARED((N_PAD, LATDIM), jnp.float32),
        ],
    )(flat_tables, edp, vals)


GB_PER_W = 10 * BATCH // (NC * NS)
GCHUNK = 128
GB_CH = GB_PER_W // GCHUNK


def _gather_body(table, idx, out, idx_v, rb0, rb1, rb2,
                 gs0, gs1, gs2, os0, os1, os2):
    c = lax.axis_index("c")
    s = lax.axis_index("s")
    w = s * NC + c
    base = w * GB_PER_W
    pltpu.sync_copy(idx.at[pl.ds(base, GB_PER_W)], idx_v)
    rbufs = (rb0, rb1, rb2)
    gsem = (gs0, gs1, gs2)
    osem = (os0, os1, os2)

    def g_start(k, b):
        pltpu.async_copy(table.at[idx_v.at[pl.ds(k * GCHUNK, GCHUNK)]],
                         rbufs[b], gsem[b])

    def g_wait(k, b):
        pltpu.make_async_copy(table.at[idx_v.at[pl.ds(k * GCHUNK, GCHUNK)]],
                              rbufs[b], gsem[b]).wait()

    def o_start(k, b):
        pltpu.async_copy(rbufs[b], out.at[pl.ds(base + k * GCHUNK, GCHUNK)],
                         osem[b])

    def o_wait(k, b):
        pltpu.make_async_copy(rbufs[b], out.at[pl.ds(base + k * GCHUNK, GCHUNK)],
                              osem[b]).wait()

    g_start(0, 0)
    for k in range(GB_CH):
        b = k % 3
        bn = (k + 1) % 3
        g_wait(k, b)
        if k >= 2:
            o_wait(k - 2, bn)
        if k + 1 < GB_CH:
            g_start(k + 1, bn)
        o_start(k, b)
    o_wait(GB_CH - 2, (GB_CH - 2) % 3)
    o_wait(GB_CH - 1, (GB_CH - 1) % 3)


@jax.jit
def _sc_gather(table, idx):
    mesh = plsc.VectorSubcoreMesh(core_axis_name="c", subcore_axis_name="s")
    return pl.kernel(
        _gather_body,
        out_type=jax.ShapeDtypeStruct((10 * BATCH, LATDIM), jnp.float32),
        mesh=mesh,
        scratch_types=[
            pltpu.VMEM((GB_PER_W,), jnp.int32),
            pltpu.VMEM((GCHUNK, LATDIM), jnp.float32),
            pltpu.VMEM((GCHUNK, LATDIM), jnp.float32),
            pltpu.VMEM((GCHUNK, LATDIM), jnp.float32),
            pltpu.SemaphoreType.DMA,
            pltpu.SemaphoreType.DMA,
            pltpu.SemaphoreType.DMA,
            pltpu.SemaphoreType.DMA,
            pltpu.SemaphoreType.DMA,
            pltpu.SemaphoreType.DMA,
        ],
    )(table, idx)


RB = 2000
NBK = N_USER // RB


def _k0_body(e_ref, hw_ref, hyp_ref, reg_ref, acc):
    d = pl.program_id(0)
    i = pl.program_id(1)

    @pl.when((d == 0) & (i == 0))
    def _():
        acc[0] = 0.0
    eb = e_ref[0]
    hyp_ref[0] = jnp.dot(eb, hw_ref[0], preferred_element_type=jnp.float32)
    part = jnp.sum(eb * eb)

    @pl.when(i == 0)
    def _():
        acc[0] += jnp.sum(hw_ref[0] * hw_ref[0])
    acc[0] += part

    @pl.when((d == 1) & (i == NBK - 1))
    def _():
        reg_ref[...] = jnp.full((8, LATDIM), acc[0], jnp.float32)


@jax.jit
def _k0(E, Hw):
    return pl.pallas_call(
        _k0_body,
        grid=(2, NBK),
        in_specs=[
            pl.BlockSpec((1, RB, LATDIM), lambda d, i: (d, i, 0)),
            pl.BlockSpec((1, LATDIM, HYPERNUM), lambda d, i: (d, 0, 0)),
        ],
        out_specs=[
            pl.BlockSpec((1, RB, HYPERNUM), lambda d, i: (d, i, 0)),
            pl.BlockSpec((8, LATDIM), lambda d, i: (0, 0)),
        ],
        out_shape=[
            jax.ShapeDtypeStruct((2, N_USER, HYPERNUM), jnp.float32),
            jax.ShapeDtypeStruct((8, LATDIM), jnp.float32),
        ],
        scratch_shapes=[pltpu.SMEM((1,), jnp.float32)],
    )(E, Hw)


def _k1_body(lats_ref, hyp_ref, w_ref, lat4_ref, acc):
    i = pl.program_id(1)

    @pl.when(i == 0)
    def _():
        acc[...] = jnp.zeros((HYPERNUM, LATDIM), jnp.float32)
    dn = (((0,), (0,)), ((), ()))
    acc[...] += lax.dot_general(hyp_ref[0], lats_ref[0], dn,
                                preferred_element_type=jnp.float32)

    @pl.when(i == NBK - 1)
    def _():
        lat1 = _lr(acc[...])
        lat2 = _lr(lax.dot_general(lat1, w_ref[0, 0], dn,
                                   preferred_element_type=jnp.float32)).T + lat1
        lat3 = _lr(lax.dot_general(lat2, w_ref[0, 1], dn,
                                   preferred_element_type=jnp.float32)).T + lat2
        lat4 = _lr(lax.dot_general(lat3, w_ref[0, 2], dn,
                                   preferred_element_type=jnp.float32)).T + lat3
        lat4_ref[0] = lat4


@jax.jit
def _k1(lats, hyp, W):
    return pl.pallas_call(
        _k1_body,
        grid=(2, NBK),
        in_specs=[
            pl.BlockSpec((1, RB, LATDIM), lambda d, i: (d, i, 0)),
            pl.BlockSpec((1, RB, HYPERNUM), lambda d, i: (d, i, 0)),
            pl.BlockSpec((1, 3, HYPERNUM, HYPERNUM), lambda d, i: (d, 0, 0, 0)),
        ],
        out_specs=pl.BlockSpec((1, HYPERNUM, LATDIM), lambda d, i: (d, 0, 0)),
        out_shape=jax.ShapeDtypeStruct((2, HYPERNUM, LATDIM), jnp.float32),
        scratch_shapes=[pltpu.VMEM((HYPERNUM, LATDIM), jnp.float32)],
    )(lats, hyp, W)


def _k2_body(hyp_ref, lat4_ref, acc_ref, prev_ref, lats0_ref,
             hypo_ref, gnn_ref, latsn_ref, sums_ref):
    h = _lr(jnp.dot(hyp_ref[0], lat4_ref[0],
                    preferred_element_type=jnp.float32))
    g = _lr(acc_ref[0])
    p = prev_ref[0]
    ln = g + h + p
    hypo_ref[0] = h
    gnn_ref[0] = g
    latsn_ref[0] = ln
    if sums_ref is not None:
        sums_ref[0] = ln + p + lats0_ref[0]


@functools.partial(jax.jit, static_argnames=("with_sums",))
def _k2(hyp, lat4, acc, prev, lats0, with_sums=False):
    n_out = 4 if with_sums else 3
    body = _k2_body if with_sums else (
        lambda a, b, c, d, e, f, g, h: _k2_body(a, b, c, d, e, f, g, h, None))
    outs = pl.pallas_call(
        body,
        grid=(2, NBK),
        in_specs=[
            pl.BlockSpec((1, RB, HYPERNUM), lambda d, i: (d, i, 0)),
            pl.BlockSpec((1, HYPERNUM, LATDIM), lambda d, i: (d, 0, 0)),
            pl.BlockSpec((1, RB, LATDIM), lambda d, i: (d, i, 0)),
            pl.BlockSpec((1, RB, LATDIM), lambda d, i: (d, i, 0)),
            pl.BlockSpec((1, RB, LATDIM), lambda d, i: (d, i, 0)),
        ],
        out_specs=[pl.BlockSpec((1, RB, LATDIM), lambda d, i: (d, i, 0))
                   for _ in range(n_out)],
        out_shape=[jax.ShapeDtypeStruct((2, N_USER, LATDIM), jnp.float32)
                   for _ in range(n_out)],
    )(hyp, lat4, acc, prev, lats0)
    return outs


def _k3a_body(gh_ref, wt_ref, h_ref):
    x = gh_ref[0]
    nrm = jnp.sqrt(jnp.sum(x * x, axis=1, keepdims=True))
    xn = x / jnp.maximum(nrm, 1e-12)
    h_ref[0] = jnp.dot(xn, wt_ref[0], preferred_element_type=jnp.float32)


@jax.jit
def _k3a(ghyp, wts):
    return pl.pallas_call(
        _k3a_body,
        grid=(4,),
        in_specs=[
            pl.BlockSpec((1, BATCH, LATDIM), lambda j: (j, 0, 0)),
            pl.BlockSpec((1, LATDIM, LATDIM), lambda j: (j, 0, 0)),
        ],
        out_specs=pl.BlockSpec((1, BATCH, LATDIM), lambda j: (j, 0, 0)),
        out_shape=jax.ShapeDtypeStruct((4, BATCH, LATDIM), jnp.float32),
    )(ghyp, wts)


SSL_RB = 512
SSL_NB = BATCH // SSL_RB


def _k3b_body(h_ref, gg_ref, m_ref, ssl_ref, acc):
    j = pl.program_id(0)
    r = pl.program_id(1)

    @pl.when((j == 0) & (r == 0))
    def _():
        acc[0] = 0.0
    x = gg_ref[0]
    nrm = jnp.sqrt(jnp.sum(x * x, axis=1, keepdims=True))
    g = x / jnp.maximum(nrm, 1e-12)
    hfull = h_ref[0]
    dn = (((1,), (1,)), ((), ()))
    S = lax.dot_general(g, hfull, dn, preferred_element_type=jnp.float32)
    m = m_ref[0, 0]
    neg = jnp.sum(jnp.exp(S) * m[None, :], axis=1)
    hblk = h_ref[0, pl.ds(r * SSL_RB, SSL_RB), :]
    pos = jnp.exp(jnp.sum(hblk * g, axis=1))
    mrow = m_ref[0, 0, pl.ds(r * SSL_RB, SSL_RB)]
    acc[0] += jnp.sum(mrow * (-jnp.log(pos / (neg + 1e-08) + 1e-08)))

    @pl.when((j == 3) & (r == SSL_NB - 1))
    def _():
        ssl_ref[...] = jnp.full((8, LATDIM), acc[0], jnp.float32)


@jax.jit
def _k3b(h, ggnn, masks3):
    return pl.pallas_call(
        _k3b_body,
        grid=(4, SSL_NB),
        in_specs=[
            pl.BlockSpec((1, BATCH, LATDIM), lambda j, r: (j, 0, 0)),
            pl.BlockSpec((1, SSL_RB, LATDIM), lambda j, r: (j, r, 0)),
            pl.BlockSpec((1, 1, BATCH), lambda j, r: (j, 0, 0)),
        ],
        out_specs=pl.BlockSpec((8, LATDIM), lambda j, r: (0, 0)),
        out_shape=jax.ShapeDtypeStruct((8, LATDIM), jnp.float32),
        scratch_shapes=[pltpu.SMEM((1,), jnp.float32)],
    )(h, ggnn, masks3)


def _k4_body(u_ref, i_ref, o_ref):
    o_ref[...] = jnp.sum(u_ref[...] * i_ref[...], axis=-1)


@jax.jit
def _k4(u3, i3):
    return pl.pallas_call(
        _k4_body,
        grid=(1,),
        in_specs=[
            pl.BlockSpec((32, LATDIM, LATDIM), lambda i: (0, 0, 0)),
            pl.BlockSpec((32, LATDIM, LATDIM), lambda i: (0, 0, 0)),
        ],
        out_specs=pl.BlockSpec((32, LATDIM), lambda i: (0, 0)),
        out_shape=jax.ShapeDtypeStruct((32, LATDIM), jnp.float32),
    )(u3, i3)


def kernel(uids, iids, edge_index, edge_vals, uEmbed0, iEmbed0, uhyper, ihyper, WU, WI, WT):
    row = edge_index[0]
    col = edge_index[1]
    uniq_u = jnp.unique(uids, size=BATCH, fill_value=0)
    uniq_i = jnp.unique(iids, size=BATCH, fill_value=0)
    num_u = 1 + jnp.sum((uniq_u[1:] > uniq_u[:-1]).astype(jnp.int32))
    num_i = 1 + jnp.sum((uniq_i[1:] > uniq_i[:-1]).astype(jnp.int32))
    mask_u = (jnp.arange(BATCH) < num_u).astype(jnp.float32)
    mask_i = (jnp.arange(BATCH) < num_i).astype(jnp.float32)

    padz = jnp.zeros((E_PAD - N_EDGES,), dtype=jnp.int32)
    colp = jnp.concatenate([col, padz]).reshape(NCHUNKS, CHUNK)
    rowp = jnp.concatenate([row, padz]).reshape(NCHUNKS, CHUNK)
    valp = jnp.concatenate([edge_vals, padz.astype(jnp.float32)]
                           ).reshape(NCHUNKS, 1, CHUNK)
    edp = jnp.stack([
        jnp.stack([colp, rowp], axis=1),
        jnp.stack([rowp + N_USER, colp], axis=1),
    ])

    E0 = jnp.stack([uEmbed0, iEmbed0])
    Hw = jnp.stack([uhyper, ihyper])
    hyp, regp = _k0(E0, Hw)
    reg = regp[0, 0]

    lats = E0
    hypos, gnns = [], []
    sums = None
    for i in range(GNN_LAYER):
        flat_tables = jnp.concatenate([lats[1], lats[0]], axis=0)
        acc = _sc_spmm(flat_tables, edp, valp)
        W = jnp.stack([WU[i], WI[i]])
        lat4 = _k1(lats, hyp, W)
        outs = _k2(hyp, lat4, acc, lats, E0, with_sums=(i == GNN_LAYER - 1))
        if i == GNN_LAYER - 1:
            hypo, gnn, lats, sums = outs
        else:
            hypo, gnn, lats = outs
        hypos.append(hypo)
        gnns.append(gnn)

    table = jnp.concatenate(
        [hypos[0].reshape(-1, LATDIM), hypos[1].reshape(-1, LATDIM),
         gnns[0].reshape(-1, LATDIM), gnns[1].reshape(-1, LATDIM),
         sums.reshape(-1, LATDIM)], axis=0)
    idx_all = jnp.concatenate([
        uniq_u, uniq_i + N_USER,
        uniq_u + 2 * N_USER, uniq_i + 3 * N_USER,
        uniq_u + 4 * N_USER, uniq_i + 5 * N_USER,
        uniq_u + 6 * N_USER, uniq_i + 7 * N_USER,
        uids + 8 * N_USER, iids + 9 * N_USER,
    ]).astype(jnp.int32)
    rowsg = _sc_gather(table, idx_all).reshape(10, BATCH, LATDIM)

    ghyp = rowsg[0:4]
    ggnn = rowsg[4:8]
    wts = jnp.stack([WT[0], WT[0], WT[1], WT[1]])
    masks3 = jnp.stack([mask_u, mask_i, mask_u, mask_i])[:, None, :]
    h = _k3a(ghyp, wts)
    sslp = _k3b(h, ggnn, masks3)
    ssl = sslp[0, 0]

    preds = _k4(rowsg[8].reshape(32, LATDIM, LATDIM),
                rowsg[9].reshape(32, LATDIM, LATDIM)).reshape(BATCH)
    return (preds, ssl, reg)
